# baseline (speedup 1.0000x reference)
import jax
import jax.numpy as jnp
from jax import lax
from jax.experimental import pallas as pl
from jax.experimental.pallas import tpu as pltpu

_N = 4096
_D = 256
_CH = 256
_RC = 512
_RC0 = 256
_M0 = 16
_M1 = 30
_NC0 = _N // _RC0
_NU0 = _N // _RC
_NC1 = (_N // 2) // _RC
_NBUF = 2


def _dot(a, b, dims, precision=lax.Precision.HIGHEST):
    return lax.dot_general(a, b, (dims, ((), ())), precision=precision,
                           preferred_element_type=jnp.float32)


def _sigmoid(x):
    e = jnp.exp(-jnp.abs(x))
    return jnp.where(x >= 0, 1.0 / (1.0 + e), e / (1.0 + e))


def _eye128():
    r = lax.broadcasted_iota(jnp.int32, (128, 128), 0)
    c = lax.broadcasted_iota(jnp.int32, (128, 128), 1)
    return jnp.where(r == c, 1.0, 0.0)


def _to_col(v, n, eye):
    cols = []
    for k in range(n // 128):
        ch = lax.slice(v, (0, k * 128), (1, (k + 1) * 128))
        cols.append(_dot(eye, ch, ((1,), (1,))))
    return jnp.concatenate(cols, axis=0)


def _s0_copy(hbm, rings, sems, c):
    ring = rings[c % 2]
    sem = sems[c % 2]
    slot = (c // 2) % _NBUF
    return pltpu.make_async_copy(
        hbm.at[pl.ds(c * _RC0, _RC0), :], ring.at[slot], sem.at[slot])


def _s1_copy(hbm, vmem, sem, c):
    return pltpu.make_async_copy(
        hbm.at[pl.ds(c * _RC, _RC), :], vmem.at[pl.ds(c * _RC, _RC), :],
        sem.at[c])


def _lse_taylor(a_row, nk, order):
    amax = jnp.max(a_row)
    safe = jnp.where(amax > 0, amax, 1.0)
    b = a_row * (1.0 / safe)
    w = jnp.exp(a_row - amax)
    t = a_row * (safe * (1.0 / nk))
    moments = []
    p = w
    for _ in range(order + 1):
        moments.append(jnp.sum(p))
        p = p * b
    s = jnp.full_like(a_row, moments[order])
    for m in range(order, 0, -1):
        s = s * (t * (1.0 / m)) + moments[m - 1]
    h = amax + jnp.log(s)
    return jnp.maximum(h, 0.0)


def _lse_exact(a_row, a_col, nk):
    amax = jnp.max(a_row)
    c = 1.0 + a_row * (1.0 / nk)
    acc = jnp.zeros((1, nk), jnp.float32)
    for ib in range(nk // _CH):
        ai = lax.slice(a_col, (ib * _CH, 0), ((ib + 1) * _CH, 1))
        e = jnp.exp((ai - amax) * c)
        acc = acc + jnp.sum(e, axis=0, keepdims=True)
    h = amax * c + jnp.log(acc)
    return jnp.maximum(h, 0.0)


def _body(x_ref, g0_ref, g1_ref, w_hbm, s0_hbm, s1_hbm, out_ref,
          w_v, s0_ring_a, s0_ring_b, s0b_v, s1_v, semw, sem0a, sem0b, sem1):
    rings = (s0_ring_a, s0_ring_b)
    sems = (sem0a, sem0b)
    w_cp = pltpu.make_async_copy(w_hbm, w_v, semw)
    w_cp.start()
    for c in range(2 * _NBUF):
        _s0_copy(s0_hbm, rings, sems, c).start()
    for c in range(_NC1):
        _s1_copy(s1_hbm, s1_v, sem1, c).start()
    eye = _eye128()
    x_col = _to_col(x_ref[...], _D, eye)
    w_cp.wait()
    a0 = jnp.maximum(
        jnp.sum(w_v[...] * x_col, axis=0, keepdims=True), 0.0)
    out0 = _lse_taylor(a0, _N, _M0)
    out0_col = _to_col(out0, _N, eye)
    acc = None
    for c in range(_NC0):
        _s0_copy(s0_hbm, rings, sems, c).wait()
        chunk = rings[c % 2][(c // 2) % _NBUF]
        part = chunk * lax.slice(out0_col, (c * _RC0, 0), ((c + 1) * _RC0, 1))
        p = jnp.sum(part, axis=0, keepdims=True)
        acc = p if acc is None else acc + p
        s0b_v[pl.ds(c * _RC0, _RC0), :] = chunk.astype(jnp.bfloat16)
        if c + 2 * _NBUF < _NC0:
            _s0_copy(s0_hbm, rings, sems, c + 2 * _NBUF).start()
    a1 = jnp.maximum(acc, 0.0)
    out1 = _lse_taylor(a1, _N // 2, _M1)
    out1_col = _to_col(out1, _N // 2, eye)
    acc = None
    for c in range(_NC1):
        _s1_copy(s1_hbm, s1_v, sem1, c).wait()
        part = (s1_v[pl.ds(c * _RC, _RC), :]
                * lax.slice(out1_col, (c * _RC, 0), ((c + 1) * _RC, 1)))
        p = jnp.sum(part, axis=0, keepdims=True)
        acc = p if acc is None else acc + p
    a2 = jnp.maximum(acc, 0.0)
    a2_col = _to_col(a2, _N // 4, eye)
    out2 = _lse_exact(a2, a2_col, _N // 4)
    up1 = jnp.concatenate(
        [_dot(out2, s1_v[pl.ds(c * _RC, _RC), :], ((1,), (1,)))
         for c in range(_NC1)], axis=1)
    out1r = out1 + _sigmoid(g1_ref[...]) * jnp.maximum(up1, 0.0)
    out1r_b = out1r.astype(jnp.bfloat16)
    up0 = jnp.concatenate(
        [_dot(out1r_b, s0b_v[pl.ds(c * _RC, _RC), :], ((1,), (1,)),
              precision=lax.Precision.DEFAULT)
         for c in range(_NU0)], axis=1)
    out_ref[...] = out0 + _sigmoid(g0_ref[...]) * jnp.maximum(up0, 0.0)


def kernel(x_seq, W_in, S_0, S_1, refine_gate_0, refine_gate_1):
    g0 = refine_gate_0.reshape(1, _N)
    g1 = refine_gate_1.reshape(1, _N // 2)
    return pl.pallas_call(
        _body,
        in_specs=[
            pl.BlockSpec(memory_space=pltpu.MemorySpace.VMEM),
            pl.BlockSpec(memory_space=pltpu.MemorySpace.VMEM),
            pl.BlockSpec(memory_space=pltpu.MemorySpace.VMEM),
            pl.BlockSpec(memory_space=pl.ANY),
            pl.BlockSpec(memory_space=pl.ANY),
            pl.BlockSpec(memory_space=pl.ANY),
        ],
        scratch_shapes=[
            pltpu.VMEM((_D, _N), jnp.float32),
            pltpu.VMEM((_NBUF, _RC0, _N // 2), jnp.float32),
            pltpu.VMEM((_NBUF, _RC0, _N // 2), jnp.float32),
            pltpu.VMEM((_N, _N // 2), jnp.bfloat16),
            pltpu.VMEM((_N // 2, _N // 4), jnp.float32),
            pltpu.SemaphoreType.DMA,
            pltpu.SemaphoreType.DMA((_NBUF,)),
            pltpu.SemaphoreType.DMA((_NBUF,)),
            pltpu.SemaphoreType.DMA((_NC1,)),
        ],
        out_shape=jax.ShapeDtypeStruct((1, _N), jnp.float32),
    )(x_seq, g0, g1, W_in, S_0, S_1)


# speedup vs baseline: 1.0337x; 1.0337x over previous
import jax
import jax.numpy as jnp
from jax import lax
from jax.experimental import pallas as pl
from jax.experimental.pallas import tpu as pltpu

_N = 4096
_D = 256
_CH = 256
_RC = 512
_RC0 = 256
_M0 = 16
_M1 = 30
_NC0 = _N // _RC0
_NU0 = _N // _RC
_NC1 = (_N // 2) // _RC
_NBUF = 2


def _dot(a, b, dims, precision=lax.Precision.HIGHEST):
    return lax.dot_general(a, b, (dims, ((), ())), precision=precision,
                           preferred_element_type=jnp.float32)


def _sigmoid(x):
    e = jnp.exp(-jnp.abs(x))
    return jnp.where(x >= 0, 1.0 / (1.0 + e), e / (1.0 + e))


def _eye128():
    r = lax.broadcasted_iota(jnp.int32, (128, 128), 0)
    c = lax.broadcasted_iota(jnp.int32, (128, 128), 1)
    return jnp.where(r == c, 1.0, 0.0)


def _to_col(v, n, eye):
    cols = []
    for k in range(n // 128):
        ch = lax.slice(v, (0, k * 128), (1, (k + 1) * 128))
        cols.append(_dot(eye, ch, ((1,), (1,))))
    return jnp.concatenate(cols, axis=0)


def _s0_copy(hbm, rings, sems, c):
    ring = rings[c % 2]
    sem = sems[c % 2]
    slot = (c // 2) % _NBUF
    return pltpu.make_async_copy(
        hbm.at[pl.ds(c * _RC0, _RC0), :], ring.at[slot], sem.at[slot])


def _s1_copy(hbm, vmem, sem, c):
    return pltpu.make_async_copy(
        hbm.at[pl.ds(c * _RC, _RC), :], vmem.at[pl.ds(c * _RC, _RC), :],
        sem.at[c])


def _lse_taylor(a_row, nk, order):
    amax = jnp.max(a_row)
    safe = jnp.where(amax > 0, amax, 1.0)
    b = a_row * (1.0 / safe)
    w = jnp.exp(a_row - amax)
    t = a_row * (safe * (1.0 / nk))
    moments = []
    p = w
    for _ in range(order + 1):
        moments.append(jnp.sum(p))
        p = p * b
    s = jnp.full_like(a_row, moments[order])
    for m in range(order, 0, -1):
        s = s * (t * (1.0 / m)) + moments[m - 1]
    h = amax + jnp.log(s)
    return jnp.maximum(h, 0.0)


def _lse_exact(a_row, a_col, nk):
    amax = jnp.max(a_row)
    c = 1.0 + a_row * (1.0 / nk)
    acc = jnp.zeros((1, nk), jnp.float32)
    for ib in range(nk // _CH):
        ai = lax.slice(a_col, (ib * _CH, 0), ((ib + 1) * _CH, 1))
        e = jnp.exp((ai - amax) * c)
        acc = acc + jnp.sum(e, axis=0, keepdims=True)
    h = amax * c + jnp.log(acc)
    return jnp.maximum(h, 0.0)


def _body(x_ref, w_ref, g0_ref, g1_ref, s0_hbm, s1_hbm, out_ref,
          s0_ring_a, s0_ring_b, s0b_v, s1_v, sem0a, sem0b, sem1):
    rings = (s0_ring_a, s0_ring_b)
    sems = (sem0a, sem0b)
    for c in range(2 * _NBUF):
        _s0_copy(s0_hbm, rings, sems, c).start()
    for c in range(_NC1):
        _s1_copy(s1_hbm, s1_v, sem1, c).start()
    eye = _eye128()
    x_col = _to_col(x_ref[...], _D, eye)
    a0 = jnp.maximum(
        jnp.sum(w_ref[...] * x_col, axis=0, keepdims=True), 0.0)
    out0 = _lse_taylor(a0, _N, _M0)
    out0_col = _to_col(out0, _N, eye)
    acc = None
    for c in range(_NC0):
        _s0_copy(s0_hbm, rings, sems, c).wait()
        chunk = rings[c % 2][(c // 2) % _NBUF]
        part = chunk * lax.slice(out0_col, (c * _RC0, 0), ((c + 1) * _RC0, 1))
        p = jnp.sum(part, axis=0, keepdims=True)
        acc = p if acc is None else acc + p
        s0b_v[pl.ds(c * _RC0, _RC0), :] = chunk.astype(jnp.bfloat16)
        if c + 2 * _NBUF < _NC0:
            _s0_copy(s0_hbm, rings, sems, c + 2 * _NBUF).start()
    a1 = jnp.maximum(acc, 0.0)
    out1 = _lse_taylor(a1, _N // 2, _M1)
    out1_col = _to_col(out1, _N // 2, eye)
    acc = None
    for c in range(_NC1):
        _s1_copy(s1_hbm, s1_v, sem1, c).wait()
        part = (s1_v[pl.ds(c * _RC, _RC), :]
                * lax.slice(out1_col, (c * _RC, 0), ((c + 1) * _RC, 1)))
        p = jnp.sum(part, axis=0, keepdims=True)
        acc = p if acc is None else acc + p
    a2 = jnp.maximum(acc, 0.0)
    a2_col = _to_col(a2, _N // 4, eye)
    out2 = _lse_exact(a2, a2_col, _N // 4)
    up1 = jnp.concatenate(
        [_dot(out2, s1_v[pl.ds(c * _RC, _RC), :], ((1,), (1,)))
         for c in range(_NC1)], axis=1)
    out1r = out1 + _sigmoid(g1_ref[...]) * jnp.maximum(up1, 0.0)
    out1r_b = out1r.astype(jnp.bfloat16)
    up0 = jnp.concatenate(
        [_dot(out1r_b, s0b_v[pl.ds(c * _RC, _RC), :], ((1,), (1,)),
              precision=lax.Precision.DEFAULT)
         for c in range(_NU0)], axis=1)
    out_ref[...] = out0 + _sigmoid(g0_ref[...]) * jnp.maximum(up0, 0.0)


def kernel(x_seq, W_in, S_0, S_1, refine_gate_0, refine_gate_1):
    g0 = refine_gate_0.reshape(1, _N)
    g1 = refine_gate_1.reshape(1, _N // 2)
    return pl.pallas_call(
        _body,
        in_specs=[
            pl.BlockSpec(memory_space=pltpu.MemorySpace.VMEM),
            pl.BlockSpec(memory_space=pltpu.MemorySpace.VMEM),
            pl.BlockSpec(memory_space=pltpu.MemorySpace.VMEM),
            pl.BlockSpec(memory_space=pltpu.MemorySpace.VMEM),
            pl.BlockSpec(memory_space=pl.ANY),
            pl.BlockSpec(memory_space=pl.ANY),
        ],
        scratch_shapes=[
            pltpu.VMEM((_NBUF, _RC0, _N // 2), jnp.float32),
            pltpu.VMEM((_NBUF, _RC0, _N // 2), jnp.float32),
            pltpu.VMEM((_N, _N // 2), jnp.bfloat16),
            pltpu.VMEM((_N // 2, _N // 4), jnp.float32),
            pltpu.SemaphoreType.DMA((_NBUF,)),
            pltpu.SemaphoreType.DMA((_NBUF,)),
            pltpu.SemaphoreType.DMA((_NC1,)),
        ],
        out_shape=jax.ShapeDtypeStruct((1, _N), jnp.float32),
    )(x_seq, W_in, g0, g1, S_0, S_1)
